# baseline (speedup 1.0000x reference)
import functools

import jax
import jax.numpy as jnp
from jax import lax
from jax.experimental import pallas as pl
from jax.experimental.pallas import tpu as pltpu
from jax.experimental.pallas import tpu_sc as plsc

NUM_CORES = 2
NUM_SUBCORES = 16
NW = NUM_CORES * NUM_SUBCORES

CHUNK = 128
TOK_PER_CHUNK = CHUNK // 2
NBUF = 8
LA = NBUF - 2

T_BLK = 512


def _transpose_body(x_ref, o_ref):
    x = x_ref[...]
    x3 = x.reshape(64, T_BLK // 2, 2)
    y0 = x3[:, :, 0].T
    y1 = x3[:, :, 1].T
    o_ref[...] = jnp.concatenate([y0, y1], axis=1)


def _repack_table(table_t):
    v64, v = table_t.shape
    grid = (v + T_BLK - 1) // T_BLK
    return pl.pallas_call(
        _transpose_body,
        grid=(grid,),
        in_specs=[pl.BlockSpec((64, T_BLK), lambda j: (0, j))],
        out_specs=pl.BlockSpec((T_BLK // 2, 128), lambda j: (j, 0)),
        out_shape=jax.ShapeDtypeStruct((v // 2, 128), jnp.float32),
    )(table_t)


def _gather_kernel_body(n_chunks, table_hbm, idx_hbm, out_hbm, *scratch):
    idx_v = scratch[0]
    rows = scratch[1:1 + NBUF]
    gsems = scratch[1 + NBUF:1 + 2 * NBUF]
    osems = scratch[1 + 2 * NBUF:1 + 3 * NBUF]

    wid = lax.axis_index("s") * NUM_CORES + lax.axis_index("c")
    pltpu.sync_copy(idx_hbm.at[wid], idx_v)
    out_base = wid * (n_chunks * TOK_PER_CHUNK)

    def start_gather(c, b):
        pltpu.async_copy(table_hbm.at[idx_v.at[c]], rows[b], gsems[b])

    def wait_gather(c, b):
        pltpu.make_async_copy(table_hbm.at[idx_v.at[c]], rows[b], gsems[b]).wait()

    def start_write(c, b):
        base = out_base + c * TOK_PER_CHUNK
        pltpu.async_copy(
            rows[b].at[pl.ds(0, TOK_PER_CHUNK)],
            out_hbm.at[pl.ds(base, TOK_PER_CHUNK), pl.ds(0, 32)], osems[b])
        pltpu.async_copy(
            rows[b].at[pl.ds(TOK_PER_CHUNK, TOK_PER_CHUNK)],
            out_hbm.at[pl.ds(base, TOK_PER_CHUNK), pl.ds(32, 32)], osems[b])

    def wait_write(b):
        pltpu.make_async_copy(
            rows[b].at[pl.ds(0, TOK_PER_CHUNK)],
            out_hbm.at[pl.ds(out_base, TOK_PER_CHUNK), pl.ds(0, 32)],
            osems[b]).wait()
        pltpu.make_async_copy(
            rows[b].at[pl.ds(TOK_PER_CHUNK, TOK_PER_CHUNK)],
            out_hbm.at[pl.ds(out_base, TOK_PER_CHUNK), pl.ds(32, 32)],
            osems[b]).wait()

    n_groups = n_chunks // NBUF

    for c in range(LA):
        start_gather(c, c)

    for c in range(NBUF):
        b = c
        wait_gather(c, b)
        start_write(c, b)
        bf = (b + LA) % NBUF
        if c >= 2:
            wait_write(bf)
        start_gather(c + LA, bf)

    def body(g, _):
        for b in range(NBUF):
            c = g * NBUF + b
            wait_gather(c, b)
            start_write(c, b)
            bf = (b + LA) % NBUF
            wait_write(bf)
            start_gather(c + LA, bf)
        return 0

    lax.fori_loop(1, n_groups - 1, body, 0)

    for c in range((n_groups - 1) * NBUF, n_chunks):
        b = c % NBUF
        wait_gather(c, b)
        start_write(c, b)
        if c + LA < n_chunks:
            bf = (b + LA) % NBUF
            wait_write(bf)
            start_gather(c + LA, bf)

    for b in range(NBUF):
        wait_write(b)


def _make_gather(b_total):
    assert b_total % (NW * TOK_PER_CHUNK) == 0
    n_chunks = b_total // (NW * TOK_PER_CHUNK)
    assert n_chunks % NBUF == 0 and n_chunks // NBUF >= 2
    mesh = plsc.VectorSubcoreMesh(
        core_axis_name="c", subcore_axis_name="s",
        num_cores=NUM_CORES, num_subcores=NUM_SUBCORES)
    return pl.kernel(
        functools.partial(_gather_kernel_body, n_chunks),
        out_type=jax.ShapeDtypeStruct((b_total, 128), jnp.float32),
        mesh=mesh,
        scratch_types=(
            [pltpu.VMEM((n_chunks, CHUNK), jnp.int32)]
            + [pltpu.VMEM((CHUNK, 32), jnp.float32) for _ in range(NBUF)]
            + [pltpu.SemaphoreType.DMA for _ in range(2 * NBUF)]
        ),
        compiler_params=pltpu.CompilerParams(use_tc_tiling_on_sc=False),
    )


def kernel(vocab_ids, table):
    bsz, seq = vocab_ids.shape
    v, d = table.shape
    b_total = bsz * seq

    packed = _repack_table(table.T)
    tbl_half = packed.reshape(2 * v, 32)

    n_chunks = b_total // (NW * TOK_PER_CHUNK)
    idx = vocab_ids.reshape(NW, n_chunks, TOK_PER_CHUNK).astype(jnp.int32)
    idx2 = jnp.concatenate([2 * idx, 2 * idx + 1], axis=2)

    out = _make_gather(b_total)(tbl_half, idx2)
    return out.reshape(bsz, seq, 128)[:, :, :d]


# speedup vs baseline: 13.2158x; 13.2158x over previous
import functools

import jax
import jax.numpy as jnp
from jax import lax
from jax.experimental import pallas as pl
from jax.experimental.pallas import tpu as pltpu
from jax.experimental.pallas import tpu_sc as plsc

NUM_CORES = 2
NUM_SUBCORES = 16
NW = NUM_CORES * NUM_SUBCORES

CHUNK = 128
TOK_PER_CHUNK = CHUNK // 2
NBUF = 8
LA = NBUF - 2

T_BLK = 1024


def _transpose_body(x_ref, o_ref):
    xt = x_ref[...].T
    h = T_BLK // 2
    o_ref[...] = jnp.concatenate([xt[:h], xt[h:]], axis=1)


def _repack_table(table_t):
    v64, v = table_t.shape
    grid = (v + T_BLK - 1) // T_BLK
    return pl.pallas_call(
        _transpose_body,
        grid=(grid,),
        in_specs=[pl.BlockSpec((64, T_BLK), lambda j: (0, j))],
        out_specs=pl.BlockSpec((T_BLK // 2, 128), lambda j: (j, 0)),
        out_shape=jax.ShapeDtypeStruct((grid * (T_BLK // 2), 128), jnp.float32),
    )(table_t)


def _gather_kernel_body(n_chunks, table_hbm, idx_hbm, out_hbm, *scratch):
    idx_v = scratch[0]
    rows = scratch[1:1 + NBUF]
    gsems = scratch[1 + NBUF:1 + 2 * NBUF]
    osems = scratch[1 + 2 * NBUF:1 + 3 * NBUF]

    wid = lax.axis_index("s") * NUM_CORES + lax.axis_index("c")
    pltpu.sync_copy(idx_hbm.at[wid], idx_v)
    out_base = wid * (n_chunks * TOK_PER_CHUNK)

    def start_gather(c, b):
        pltpu.async_copy(table_hbm.at[idx_v.at[c]], rows[b], gsems[b])

    def wait_gather(c, b):
        pltpu.make_async_copy(table_hbm.at[idx_v.at[c]], rows[b], gsems[b]).wait()

    def start_write(c, b):
        base = out_base + c * TOK_PER_CHUNK
        pltpu.async_copy(
            rows[b].at[pl.ds(0, TOK_PER_CHUNK)],
            out_hbm.at[pl.ds(base, TOK_PER_CHUNK), pl.ds(0, 32)], osems[b])
        pltpu.async_copy(
            rows[b].at[pl.ds(TOK_PER_CHUNK, TOK_PER_CHUNK)],
            out_hbm.at[pl.ds(base, TOK_PER_CHUNK), pl.ds(32, 32)], osems[b])

    def wait_write(b):
        pltpu.make_async_copy(
            rows[b].at[pl.ds(0, TOK_PER_CHUNK)],
            out_hbm.at[pl.ds(out_base, TOK_PER_CHUNK), pl.ds(0, 32)],
            osems[b]).wait()
        pltpu.make_async_copy(
            rows[b].at[pl.ds(TOK_PER_CHUNK, TOK_PER_CHUNK)],
            out_hbm.at[pl.ds(out_base, TOK_PER_CHUNK), pl.ds(32, 32)],
            osems[b]).wait()

    n_groups = n_chunks // NBUF

    for c in range(LA):
        start_gather(c, c)

    for c in range(NBUF):
        b = c
        wait_gather(c, b)
        start_write(c, b)
        bf = (b + LA) % NBUF
        if c >= 2:
            wait_write(bf)
        start_gather(c + LA, bf)

    def body(g, _):
        for b in range(NBUF):
            c = g * NBUF + b
            wait_gather(c, b)
            start_write(c, b)
            bf = (b + LA) % NBUF
            wait_write(bf)
            start_gather(c + LA, bf)
        return 0

    lax.fori_loop(1, n_groups - 1, body, 0)

    for c in range((n_groups - 1) * NBUF, n_chunks):
        b = c % NBUF
        wait_gather(c, b)
        start_write(c, b)
        if c + LA < n_chunks:
            bf = (b + LA) % NBUF
            wait_write(bf)
            start_gather(c + LA, bf)

    for b in range(NBUF):
        wait_write(b)


def _make_gather(b_total):
    assert b_total % (NW * TOK_PER_CHUNK) == 0
    n_chunks = b_total // (NW * TOK_PER_CHUNK)
    assert n_chunks % NBUF == 0 and n_chunks // NBUF >= 2
    mesh = plsc.VectorSubcoreMesh(
        core_axis_name="c", subcore_axis_name="s",
        num_cores=NUM_CORES, num_subcores=NUM_SUBCORES)
    return pl.kernel(
        functools.partial(_gather_kernel_body, n_chunks),
        out_type=jax.ShapeDtypeStruct((b_total, 128), jnp.float32),
        mesh=mesh,
        scratch_types=(
            [pltpu.VMEM((n_chunks, CHUNK), jnp.int32)]
            + [pltpu.VMEM((CHUNK, 32), jnp.float32) for _ in range(NBUF)]
            + [pltpu.SemaphoreType.DMA for _ in range(2 * NBUF)]
        ),
        compiler_params=pltpu.CompilerParams(use_tc_tiling_on_sc=False),
    )


def kernel(vocab_ids, table):
    bsz, seq = vocab_ids.shape
    v, d = table.shape
    b_total = bsz * seq

    packed = _repack_table(table.T)
    tbl_half = packed.reshape(-1, 32)

    n_chunks = b_total // (NW * TOK_PER_CHUNK)
    idx = vocab_ids.reshape(NW, n_chunks, TOK_PER_CHUNK).astype(jnp.int32)
    p0 = (((idx >> 10) << 11) + ((idx & 511) << 2)
          + (((idx >> 9) & 1) << 1))
    idx2 = jnp.concatenate([p0, p0 + 1], axis=2)

    out = _make_gather(b_total)(tbl_half, idx2)
    return out.reshape(bsz, seq, 128)[:, :, :d]


# speedup vs baseline: 19.9020x; 1.5059x over previous
import functools

import jax
import jax.numpy as jnp
from jax import lax
from jax.experimental import pallas as pl
from jax.experimental.pallas import tpu as pltpu
from jax.experimental.pallas import tpu_sc as plsc

NUM_CORES = 2
NUM_SUBCORES = 16
NW = NUM_CORES * NUM_SUBCORES

CHUNK = 128
TOK_PER_CHUNK = CHUNK // 2
NBUF = 8
LA = NBUF - 2

T_BLK = 4096


def _transpose_body(x_ref, o_ref):
    xt = x_ref[...].T
    h = T_BLK // 2
    o_ref[...] = jnp.concatenate([xt[:h], xt[h:]], axis=1)


def _repack_table(table_t):
    v64, v = table_t.shape
    grid = (v + T_BLK - 1) // T_BLK
    return pl.pallas_call(
        _transpose_body,
        grid=(grid,),
        in_specs=[pl.BlockSpec((64, T_BLK), lambda j: (0, j))],
        out_specs=pl.BlockSpec((T_BLK // 2, 128), lambda j: (j, 0)),
        out_shape=jax.ShapeDtypeStruct((grid * (T_BLK // 2), 128), jnp.float32),
    )(table_t)


def _gather_kernel_body(n_chunks, table_hbm, idx_hbm, out_hbm, *scratch):
    idx_v = scratch[0]
    rows = scratch[1:1 + NBUF]
    gsems = scratch[1 + NBUF:1 + 2 * NBUF]
    osems = scratch[1 + 2 * NBUF:1 + 3 * NBUF]

    wid = lax.axis_index("s") * NUM_CORES + lax.axis_index("c")
    pltpu.sync_copy(idx_hbm.at[wid], idx_v)
    out_base = wid * (n_chunks * TOK_PER_CHUNK)

    def start_gather(c, b):
        pltpu.async_copy(table_hbm.at[idx_v.at[c]], rows[b], gsems[b])

    def wait_gather(c, b):
        pltpu.make_async_copy(table_hbm.at[idx_v.at[c]], rows[b], gsems[b]).wait()

    def start_write(c, b):
        base = out_base + c * TOK_PER_CHUNK
        pltpu.async_copy(
            rows[b].at[pl.ds(0, TOK_PER_CHUNK)],
            out_hbm.at[pl.ds(base, TOK_PER_CHUNK), pl.ds(0, 32)], osems[b])
        pltpu.async_copy(
            rows[b].at[pl.ds(TOK_PER_CHUNK, TOK_PER_CHUNK)],
            out_hbm.at[pl.ds(base, TOK_PER_CHUNK), pl.ds(32, 32)], osems[b])

    def wait_write(b):
        pltpu.make_async_copy(
            rows[b].at[pl.ds(0, TOK_PER_CHUNK)],
            out_hbm.at[pl.ds(out_base, TOK_PER_CHUNK), pl.ds(0, 32)],
            osems[b]).wait()
        pltpu.make_async_copy(
            rows[b].at[pl.ds(TOK_PER_CHUNK, TOK_PER_CHUNK)],
            out_hbm.at[pl.ds(out_base, TOK_PER_CHUNK), pl.ds(32, 32)],
            osems[b]).wait()

    n_groups = n_chunks // NBUF

    for c in range(LA):
        start_gather(c, c)

    for c in range(NBUF):
        b = c
        wait_gather(c, b)
        start_write(c, b)
        bf = (b + LA) % NBUF
        if c >= 2:
            wait_write(bf)
        start_gather(c + LA, bf)

    def body(g, _):
        for b in range(NBUF):
            c = g * NBUF + b
            wait_gather(c, b)
            start_write(c, b)
            bf = (b + LA) % NBUF
            wait_write(bf)
            start_gather(c + LA, bf)
        return 0

    lax.fori_loop(1, n_groups - 1, body, 0)

    for c in range((n_groups - 1) * NBUF, n_chunks):
        b = c % NBUF
        wait_gather(c, b)
        start_write(c, b)
        if c + LA < n_chunks:
            bf = (b + LA) % NBUF
            wait_write(bf)
            start_gather(c + LA, bf)

    for b in range(NBUF):
        wait_write(b)


def _make_gather(b_total):
    assert b_total % (NW * TOK_PER_CHUNK) == 0
    n_chunks = b_total // (NW * TOK_PER_CHUNK)
    assert n_chunks % NBUF == 0 and n_chunks // NBUF >= 2
    mesh = plsc.VectorSubcoreMesh(
        core_axis_name="c", subcore_axis_name="s",
        num_cores=NUM_CORES, num_subcores=NUM_SUBCORES)
    return pl.kernel(
        functools.partial(_gather_kernel_body, n_chunks),
        out_type=jax.ShapeDtypeStruct((b_total, 128), jnp.float32),
        mesh=mesh,
        scratch_types=(
            [pltpu.VMEM((n_chunks, CHUNK), jnp.int32)]
            + [pltpu.VMEM((CHUNK, 32), jnp.float32) for _ in range(NBUF)]
            + [pltpu.SemaphoreType.DMA for _ in range(2 * NBUF)]
        ),
        compiler_params=pltpu.CompilerParams(use_tc_tiling_on_sc=False),
    )


def kernel(vocab_ids, table):
    bsz, seq = vocab_ids.shape
    v, d = table.shape
    b_total = bsz * seq

    packed = _repack_table(table.T)
    tbl_half = packed.reshape(-1, 32)

    n_chunks = b_total // (NW * TOK_PER_CHUNK)
    idx = vocab_ids.reshape(NW, n_chunks, TOK_PER_CHUNK).astype(jnp.int32)
    h = T_BLK // 2
    p0 = ((idx // T_BLK) * (4 * h) + (idx % h) * 4
          + ((idx % T_BLK) // h) * 2)
    idx2 = jnp.concatenate([p0, p0 + 1], axis=2)

    out = _make_gather(b_total)(tbl_half, idx2)
    return out.reshape(bsz, seq, 128)[:, :, :d]


# speedup vs baseline: 22.0321x; 1.1070x over previous
import functools

import jax
import jax.numpy as jnp
from jax import lax
from jax.experimental import pallas as pl
from jax.experimental.pallas import tpu as pltpu
from jax.experimental.pallas import tpu_sc as plsc

NUM_CORES = 2
NUM_SUBCORES = 16
NW = NUM_CORES * NUM_SUBCORES

CHUNK = 128
TOK_PER_CHUNK = CHUNK // 2
NBUF = 8
LA = NBUF - 2

T_BLK = 8192


def _transpose_body(x_ref, o_ref):
    xt = x_ref[...].T
    h = T_BLK // 2
    o_ref[...] = jnp.concatenate([xt[:h], xt[h:]], axis=1)


def _repack_table(table_t):
    v64, v = table_t.shape
    grid = (v + T_BLK - 1) // T_BLK
    return pl.pallas_call(
        _transpose_body,
        grid=(grid,),
        in_specs=[pl.BlockSpec((64, T_BLK), lambda j: (0, j))],
        out_specs=pl.BlockSpec((T_BLK // 2, 128), lambda j: (j, 0)),
        out_shape=jax.ShapeDtypeStruct((grid * (T_BLK // 2), 128), jnp.float32),
    )(table_t)


def _gather_kernel_body(n_chunks, table_hbm, idx_hbm, out_hbm, *scratch):
    idx_v = scratch[0]
    rows = scratch[1:1 + NBUF]
    gsems = scratch[1 + NBUF:1 + 2 * NBUF]
    osems = scratch[1 + 2 * NBUF:1 + 3 * NBUF]

    wid = lax.axis_index("s") * NUM_CORES + lax.axis_index("c")
    pltpu.sync_copy(idx_hbm.at[wid], idx_v)
    out_base = wid * (n_chunks * TOK_PER_CHUNK)

    def start_gather(c, b):
        pltpu.async_copy(table_hbm.at[idx_v.at[c]], rows[b], gsems[b])

    def wait_gather(c, b):
        pltpu.make_async_copy(table_hbm.at[idx_v.at[c]], rows[b], gsems[b]).wait()

    def start_write(c, b):
        base = out_base + c * TOK_PER_CHUNK
        pltpu.async_copy(
            rows[b].at[pl.ds(0, TOK_PER_CHUNK)],
            out_hbm.at[pl.ds(base, TOK_PER_CHUNK), pl.ds(0, 32)], osems[b])
        pltpu.async_copy(
            rows[b].at[pl.ds(TOK_PER_CHUNK, TOK_PER_CHUNK)],
            out_hbm.at[pl.ds(base, TOK_PER_CHUNK), pl.ds(32, 32)], osems[b])

    def wait_write(b):
        pltpu.make_async_copy(
            rows[b].at[pl.ds(0, TOK_PER_CHUNK)],
            out_hbm.at[pl.ds(out_base, TOK_PER_CHUNK), pl.ds(0, 32)],
            osems[b]).wait()
        pltpu.make_async_copy(
            rows[b].at[pl.ds(TOK_PER_CHUNK, TOK_PER_CHUNK)],
            out_hbm.at[pl.ds(out_base, TOK_PER_CHUNK), pl.ds(32, 32)],
            osems[b]).wait()

    n_groups = n_chunks // NBUF

    for c in range(LA):
        start_gather(c, c)

    for c in range(NBUF):
        b = c
        wait_gather(c, b)
        start_write(c, b)
        bf = (b + LA) % NBUF
        if c >= 2:
            wait_write(bf)
        start_gather(c + LA, bf)

    def body(g, _):
        for b in range(NBUF):
            c = g * NBUF + b
            wait_gather(c, b)
            start_write(c, b)
            bf = (b + LA) % NBUF
            wait_write(bf)
            start_gather(c + LA, bf)
        return 0

    lax.fori_loop(1, n_groups - 1, body, 0)

    for c in range((n_groups - 1) * NBUF, n_chunks):
        b = c % NBUF
        wait_gather(c, b)
        start_write(c, b)
        if c + LA < n_chunks:
            bf = (b + LA) % NBUF
            wait_write(bf)
            start_gather(c + LA, bf)

    for b in range(NBUF):
        wait_write(b)


def _make_gather(b_total):
    assert b_total % (NW * TOK_PER_CHUNK) == 0
    n_chunks = b_total // (NW * TOK_PER_CHUNK)
    assert n_chunks % NBUF == 0 and n_chunks // NBUF >= 2
    mesh = plsc.VectorSubcoreMesh(
        core_axis_name="c", subcore_axis_name="s",
        num_cores=NUM_CORES, num_subcores=NUM_SUBCORES)
    return pl.kernel(
        functools.partial(_gather_kernel_body, n_chunks),
        out_type=jax.ShapeDtypeStruct((b_total, 128), jnp.float32),
        mesh=mesh,
        scratch_types=(
            [pltpu.VMEM((n_chunks, CHUNK), jnp.int32)]
            + [pltpu.VMEM((CHUNK, 32), jnp.float32) for _ in range(NBUF)]
            + [pltpu.SemaphoreType.DMA for _ in range(2 * NBUF)]
        ),
        compiler_params=pltpu.CompilerParams(use_tc_tiling_on_sc=False),
    )


def kernel(vocab_ids, table):
    bsz, seq = vocab_ids.shape
    v, d = table.shape
    b_total = bsz * seq

    packed = _repack_table(table.T)
    tbl_half = packed.reshape(-1, 32)

    n_chunks = b_total // (NW * TOK_PER_CHUNK)
    idx = vocab_ids.reshape(NW, n_chunks, TOK_PER_CHUNK).astype(jnp.int32)
    h = T_BLK // 2
    p0 = ((idx // T_BLK) * (4 * h) + (idx % h) * 4
          + ((idx % T_BLK) // h) * 2)
    idx2 = jnp.concatenate([p0, p0 + 1], axis=2)

    out = _make_gather(b_total)(tbl_half, idx2)
    return out.reshape(bsz, seq, 128)[:, :, :d]


# speedup vs baseline: 23.2472x; 1.0552x over previous
import functools

import jax
import jax.numpy as jnp
from jax import lax
from jax.experimental import pallas as pl
from jax.experimental.pallas import tpu as pltpu
from jax.experimental.pallas import tpu_sc as plsc

NUM_CORES = 2
NUM_SUBCORES = 16
NW = NUM_CORES * NUM_SUBCORES

CHUNK = 128
TOK_PER_CHUNK = CHUNK // 2
NBUF = 8
LA = NBUF - 2

T_BLK = 16384


def _transpose_body(x_ref, o_ref):
    xt = x_ref[...].T
    h = T_BLK // 2
    o_ref[...] = jnp.concatenate([xt[:h], xt[h:]], axis=1)


def _repack_table(table_t):
    v64, v = table_t.shape
    grid = (v + T_BLK - 1) // T_BLK
    return pl.pallas_call(
        _transpose_body,
        grid=(grid,),
        in_specs=[pl.BlockSpec((64, T_BLK), lambda j: (0, j))],
        out_specs=pl.BlockSpec((T_BLK // 2, 128), lambda j: (j, 0)),
        out_shape=jax.ShapeDtypeStruct((grid * (T_BLK // 2), 128), jnp.float32),
    )(table_t)


def _gather_kernel_body(n_chunks, table_hbm, idx_hbm, out_hbm, *scratch):
    idx_v = scratch[0]
    rows = scratch[1:1 + NBUF]
    gsems = scratch[1 + NBUF:1 + 2 * NBUF]
    osems = scratch[1 + 2 * NBUF:1 + 3 * NBUF]

    wid = lax.axis_index("s") * NUM_CORES + lax.axis_index("c")
    pltpu.sync_copy(idx_hbm.at[wid], idx_v)
    out_base = wid * (n_chunks * TOK_PER_CHUNK)

    def start_gather(c, b):
        pltpu.async_copy(table_hbm.at[idx_v.at[c]], rows[b], gsems[b])

    def wait_gather(c, b):
        pltpu.make_async_copy(table_hbm.at[idx_v.at[c]], rows[b], gsems[b]).wait()

    def start_write(c, b):
        base = out_base + c * TOK_PER_CHUNK
        pltpu.async_copy(
            rows[b].at[pl.ds(0, TOK_PER_CHUNK)],
            out_hbm.at[pl.ds(base, TOK_PER_CHUNK), pl.ds(0, 32)], osems[b])
        pltpu.async_copy(
            rows[b].at[pl.ds(TOK_PER_CHUNK, TOK_PER_CHUNK)],
            out_hbm.at[pl.ds(base, TOK_PER_CHUNK), pl.ds(32, 32)], osems[b])

    def wait_write(b):
        pltpu.make_async_copy(
            rows[b].at[pl.ds(0, TOK_PER_CHUNK)],
            out_hbm.at[pl.ds(out_base, TOK_PER_CHUNK), pl.ds(0, 32)],
            osems[b]).wait()
        pltpu.make_async_copy(
            rows[b].at[pl.ds(TOK_PER_CHUNK, TOK_PER_CHUNK)],
            out_hbm.at[pl.ds(out_base, TOK_PER_CHUNK), pl.ds(32, 32)],
            osems[b]).wait()

    n_groups = n_chunks // NBUF

    for c in range(LA):
        start_gather(c, c)

    for c in range(NBUF):
        b = c
        wait_gather(c, b)
        start_write(c, b)
        bf = (b + LA) % NBUF
        if c >= 2:
            wait_write(bf)
        start_gather(c + LA, bf)

    def body(g, _):
        for b in range(NBUF):
            c = g * NBUF + b
            wait_gather(c, b)
            start_write(c, b)
            bf = (b + LA) % NBUF
            wait_write(bf)
            start_gather(c + LA, bf)
        return 0

    lax.fori_loop(1, n_groups - 1, body, 0)

    for c in range((n_groups - 1) * NBUF, n_chunks):
        b = c % NBUF
        wait_gather(c, b)
        start_write(c, b)
        if c + LA < n_chunks:
            bf = (b + LA) % NBUF
            wait_write(bf)
            start_gather(c + LA, bf)

    for b in range(NBUF):
        wait_write(b)


def _make_gather(b_total):
    assert b_total % (NW * TOK_PER_CHUNK) == 0
    n_chunks = b_total // (NW * TOK_PER_CHUNK)
    assert n_chunks % NBUF == 0 and n_chunks // NBUF >= 2
    mesh = plsc.VectorSubcoreMesh(
        core_axis_name="c", subcore_axis_name="s",
        num_cores=NUM_CORES, num_subcores=NUM_SUBCORES)
    return pl.kernel(
        functools.partial(_gather_kernel_body, n_chunks),
        out_type=jax.ShapeDtypeStruct((b_total, 128), jnp.float32),
        mesh=mesh,
        scratch_types=(
            [pltpu.VMEM((n_chunks, CHUNK), jnp.int32)]
            + [pltpu.VMEM((CHUNK, 32), jnp.float32) for _ in range(NBUF)]
            + [pltpu.SemaphoreType.DMA for _ in range(2 * NBUF)]
        ),
        compiler_params=pltpu.CompilerParams(use_tc_tiling_on_sc=False),
    )


def kernel(vocab_ids, table):
    bsz, seq = vocab_ids.shape
    v, d = table.shape
    b_total = bsz * seq

    packed = _repack_table(table.T)
    tbl_half = packed.reshape(-1, 32)

    n_chunks = b_total // (NW * TOK_PER_CHUNK)
    idx = vocab_ids.reshape(NW, n_chunks, TOK_PER_CHUNK).astype(jnp.int32)
    h = T_BLK // 2
    p0 = ((idx // T_BLK) * (4 * h) + (idx % h) * 4
          + ((idx % T_BLK) // h) * 2)
    idx2 = jnp.concatenate([p0, p0 + 1], axis=2)

    out = _make_gather(b_total)(tbl_half, idx2)
    return out.reshape(bsz, seq, 128)[:, :, :d]


# speedup vs baseline: 23.8634x; 1.0265x over previous
import functools

import jax
import jax.numpy as jnp
from jax import lax
from jax.experimental import pallas as pl
from jax.experimental.pallas import tpu as pltpu
from jax.experimental.pallas import tpu_sc as plsc

NUM_CORES = 2
NUM_SUBCORES = 16
NW = NUM_CORES * NUM_SUBCORES

CHUNK = 128
TOK_PER_CHUNK = CHUNK // 2
NBUF = 8
LA = NBUF - 2

T_BLK = 32768


def _transpose_body(x_ref, o_ref):
    xt = x_ref[...].T
    h = T_BLK // 2
    o_ref[...] = jnp.concatenate([xt[:h], xt[h:]], axis=1)


def _repack_table(table_t):
    v64, v = table_t.shape
    grid = (v + T_BLK - 1) // T_BLK
    return pl.pallas_call(
        _transpose_body,
        grid=(grid,),
        in_specs=[pl.BlockSpec((64, T_BLK), lambda j: (0, j))],
        out_specs=pl.BlockSpec((T_BLK // 2, 128), lambda j: (j, 0)),
        out_shape=jax.ShapeDtypeStruct((grid * (T_BLK // 2), 128), jnp.float32),
    )(table_t)


def _gather_kernel_body(n_chunks, table_hbm, idx_hbm, out_hbm, *scratch):
    idx_v = scratch[0]
    rows = scratch[1:1 + NBUF]
    gsems = scratch[1 + NBUF:1 + 2 * NBUF]
    osems = scratch[1 + 2 * NBUF:1 + 3 * NBUF]

    wid = lax.axis_index("s") * NUM_CORES + lax.axis_index("c")
    pltpu.sync_copy(idx_hbm.at[wid], idx_v)
    out_base = wid * (n_chunks * TOK_PER_CHUNK)

    def start_gather(c, b):
        pltpu.async_copy(table_hbm.at[idx_v.at[c]], rows[b], gsems[b])

    def wait_gather(c, b):
        pltpu.make_async_copy(table_hbm.at[idx_v.at[c]], rows[b], gsems[b]).wait()

    def start_write(c, b):
        base = out_base + c * TOK_PER_CHUNK
        pltpu.async_copy(
            rows[b].at[pl.ds(0, TOK_PER_CHUNK)],
            out_hbm.at[pl.ds(base, TOK_PER_CHUNK), pl.ds(0, 32)], osems[b])
        pltpu.async_copy(
            rows[b].at[pl.ds(TOK_PER_CHUNK, TOK_PER_CHUNK)],
            out_hbm.at[pl.ds(base, TOK_PER_CHUNK), pl.ds(32, 32)], osems[b])

    def wait_write(b):
        pltpu.make_async_copy(
            rows[b].at[pl.ds(0, TOK_PER_CHUNK)],
            out_hbm.at[pl.ds(out_base, TOK_PER_CHUNK), pl.ds(0, 32)],
            osems[b]).wait()
        pltpu.make_async_copy(
            rows[b].at[pl.ds(TOK_PER_CHUNK, TOK_PER_CHUNK)],
            out_hbm.at[pl.ds(out_base, TOK_PER_CHUNK), pl.ds(32, 32)],
            osems[b]).wait()

    n_groups = n_chunks // NBUF

    for c in range(LA):
        start_gather(c, c)

    for c in range(NBUF):
        b = c
        wait_gather(c, b)
        start_write(c, b)
        bf = (b + LA) % NBUF
        if c >= 2:
            wait_write(bf)
        start_gather(c + LA, bf)

    def body(g, _):
        for b in range(NBUF):
            c = g * NBUF + b
            wait_gather(c, b)
            start_write(c, b)
            bf = (b + LA) % NBUF
            wait_write(bf)
            start_gather(c + LA, bf)
        return 0

    lax.fori_loop(1, n_groups - 1, body, 0)

    for c in range((n_groups - 1) * NBUF, n_chunks):
        b = c % NBUF
        wait_gather(c, b)
        start_write(c, b)
        if c + LA < n_chunks:
            bf = (b + LA) % NBUF
            wait_write(bf)
            start_gather(c + LA, bf)

    for b in range(NBUF):
        wait_write(b)


def _make_gather(b_total):
    assert b_total % (NW * TOK_PER_CHUNK) == 0
    n_chunks = b_total // (NW * TOK_PER_CHUNK)
    assert n_chunks % NBUF == 0 and n_chunks // NBUF >= 2
    mesh = plsc.VectorSubcoreMesh(
        core_axis_name="c", subcore_axis_name="s",
        num_cores=NUM_CORES, num_subcores=NUM_SUBCORES)
    return pl.kernel(
        functools.partial(_gather_kernel_body, n_chunks),
        out_type=jax.ShapeDtypeStruct((b_total, 128), jnp.float32),
        mesh=mesh,
        scratch_types=(
            [pltpu.VMEM((n_chunks, CHUNK), jnp.int32)]
            + [pltpu.VMEM((CHUNK, 32), jnp.float32) for _ in range(NBUF)]
            + [pltpu.SemaphoreType.DMA for _ in range(2 * NBUF)]
        ),
        compiler_params=pltpu.CompilerParams(use_tc_tiling_on_sc=False),
    )


def kernel(vocab_ids, table):
    bsz, seq = vocab_ids.shape
    v, d = table.shape
    b_total = bsz * seq

    packed = _repack_table(table.T)
    tbl_half = packed.reshape(-1, 32)

    n_chunks = b_total // (NW * TOK_PER_CHUNK)
    idx = vocab_ids.reshape(NW, n_chunks, TOK_PER_CHUNK).astype(jnp.int32)
    h = T_BLK // 2
    p0 = ((idx // T_BLK) * (4 * h) + (idx % h) * 4
          + ((idx % T_BLK) // h) * 2)
    idx2 = jnp.concatenate([p0, p0 + 1], axis=2)

    out = _make_gather(b_total)(tbl_half, idx2)
    return out.reshape(bsz, seq, 128)[:, :, :d]
